# Initial kernel scaffold; baseline (speedup 1.0000x reference)
import jax
import jax.numpy as jnp
from jax.experimental import pallas as pl


def kernel(prior_mean, prior_var, obs_variance, x, idx_tensor):
    raise NotImplementedError("write your pallas kernel here")


# speedup vs baseline: 1.4800x; 1.4800x over previous
import functools

import jax
import jax.numpy as jnp
from jax import lax
from jax.experimental import pallas as pl
from jax.experimental.pallas import tpu as pltpu
from jax.experimental.pallas import tpu_sc as plsc

M = 1_000_000
B = 16_384
NC = 2
NS = 16
HALF = M // NC
ACC = HALF + 8
CAT = 31_248
CH = 10_416
TAIL = 32
OB = B // NS
ZCH = 4_096


def _body(pm_hbm, pv_hbm, po_hbm, x_hbm, idx_hbm, out0_hbm, out1_hbm,
          idx_buf, x_buf, x2d, loc2d, ones2d, z_buf, po_buf,
          pm_buf, pv_buf, c_buf, s_buf, cnt_sh, sum_sh):
    c = lax.axis_index("c")
    s = lax.axis_index("s")

    pltpu.sync_copy(po_hbm, po_buf)
    pltpu.sync_copy(idx_hbm.at[pl.ds(s * OB, OB)], idx_buf)

    zeros16 = jnp.zeros((16,), jnp.float32)

    def zb(i, carry):
        z_buf[pl.ds(i * 16, 16)] = zeros16
        return carry

    lax.fori_loop(0, ZCH // 16, zb, 0)

    base = c * HALF
    ones16 = jnp.ones((16,), jnp.float32)

    def lb(i, carry):
        row = i >> 3
        col = (i & 7) * 16
        iv = idx_buf[pl.ds(i * 16, 16)]
        loc = iv - base
        in_core = (loc >= 0) & (loc < HALF)
        loc2d[row, pl.ds(col, 16)] = jnp.where(in_core, loc, HALF)
        ones2d[row, pl.ds(col, 16)] = ones16
        return carry

    lax.fori_loop(0, OB // 16, lb, 0)

    pltpu.sync_copy(x_hbm.at[pl.ds(s * OB, OB)], x_buf)

    def xb(i, carry):
        row = i >> 3
        col = (i & 7) * 16
        x2d[row, pl.ds(col, 16)] = x_buf[pl.ds(i * 16, 16)]
        return carry

    lax.fori_loop(0, OB // 16, xb, 0)

    zoff = s * CAT
    for k in range(CAT // ZCH):
        for sh in (cnt_sh, sum_sh):
            pltpu.sync_copy(z_buf, sh.at[pl.ds(zoff + k * ZCH, ZCH)])
    rem = CAT - (CAT // ZCH) * ZCH
    for sh in (cnt_sh, sum_sh):
        pltpu.sync_copy(z_buf.at[pl.ds(0, rem)], sh.at[pl.ds(zoff + (CAT // ZCH) * ZCH, rem)])

    @pl.when(s == NS - 1)
    def _ztail():
        zt = NS * CAT
        for sh in (cnt_sh, sum_sh):
            pltpu.sync_copy(z_buf.at[pl.ds(0, ACC - zt)], sh.at[pl.ds(zt, ACC - zt)])

    plsc.subcore_barrier()

    for j in range(OB // 128):
        pltpu.sync_copy(ones2d.at[j], cnt_sh.at[loc2d.at[j]], add=True)
        pltpu.sync_copy(x2d.at[j], sum_sh.at[loc2d.at[j]], add=True)

    plsc.subcore_barrier()

    po = po_buf[...]

    def compute(n_vecs):
        def body(i, carry):
            sl = pl.ds(i * 16, 16)
            pm = pm_buf[sl]
            pv = pv_buf[sl]
            cnt = c_buf[sl]
            sx = s_buf[sl]
            p0 = 1.0 / pv
            pn = p0 + cnt * po
            pm_buf[sl] = (pm * p0 + sx * po) / pn
            pv_buf[sl] = 1.0 / pn
            return carry
        lax.fori_loop(0, n_vecs, body, 0)

    cat0 = s * CAT
    for k in range(CAT // CH):
        off = cat0 + k * CH
        g = base + off
        pltpu.sync_copy(pm_hbm.at[pl.ds(g, CH)], pm_buf)
        pltpu.sync_copy(pv_hbm.at[pl.ds(g, CH)], pv_buf)
        pltpu.sync_copy(cnt_sh.at[pl.ds(off, CH)], c_buf)
        pltpu.sync_copy(sum_sh.at[pl.ds(off, CH)], s_buf)
        compute(CH // 16)
        pltpu.sync_copy(pm_buf, out0_hbm.at[pl.ds(g, CH)])
        pltpu.sync_copy(pv_buf, out1_hbm.at[pl.ds(g, CH)])

    @pl.when(s == NS - 1)
    def _tail():
        off = NS * CAT
        g = base + off
        pltpu.sync_copy(pm_hbm.at[pl.ds(g, TAIL)], pm_buf.at[pl.ds(0, TAIL)])
        pltpu.sync_copy(pv_hbm.at[pl.ds(g, TAIL)], pv_buf.at[pl.ds(0, TAIL)])
        pltpu.sync_copy(cnt_sh.at[pl.ds(off, TAIL)], c_buf.at[pl.ds(0, TAIL)])
        pltpu.sync_copy(sum_sh.at[pl.ds(off, TAIL)], s_buf.at[pl.ds(0, TAIL)])
        compute(TAIL // 16)
        pltpu.sync_copy(pm_buf.at[pl.ds(0, TAIL)], out0_hbm.at[pl.ds(g, TAIL)])
        pltpu.sync_copy(pv_buf.at[pl.ds(0, TAIL)], out1_hbm.at[pl.ds(g, TAIL)])


def kernel(prior_mean, prior_var, obs_variance, x, idx_tensor):
    po_vec = jnp.full((16,), 1.0, jnp.float32) / obs_variance

    mesh = plsc.VectorSubcoreMesh(core_axis_name="c", subcore_axis_name="s")
    run = pl.kernel(
        _body,
        out_type=(
            jax.ShapeDtypeStruct((M,), jnp.float32),
            jax.ShapeDtypeStruct((M,), jnp.float32),
        ),
        mesh=mesh,
        scratch_types=(
            pltpu.VMEM((OB,), jnp.int32),
            pltpu.VMEM((OB,), jnp.float32),
            pltpu.VMEM((OB // 128, 128), jnp.float32),
            pltpu.VMEM((OB // 128, 128), jnp.int32),
            pltpu.VMEM((OB // 128, 128), jnp.float32),
            pltpu.VMEM((ZCH,), jnp.float32),
            pltpu.VMEM((16,), jnp.float32),
            pltpu.VMEM((CH,), jnp.float32),
            pltpu.VMEM((CH,), jnp.float32),
            pltpu.VMEM((CH,), jnp.float32),
            pltpu.VMEM((CH,), jnp.float32),
            pltpu.VMEM_SHARED((ACC,), jnp.float32),
            pltpu.VMEM_SHARED((ACC,), jnp.float32),
        ),
    )
    o0, o1 = run(prior_mean, prior_var, po_vec, x, idx_tensor)
    return jnp.stack([o0, o1])


# speedup vs baseline: 2.0207x; 1.3653x over previous
import functools

import jax
import jax.numpy as jnp
from jax import lax
from jax.experimental import pallas as pl
from jax.experimental.pallas import tpu as pltpu
from jax.experimental.pallas import tpu_sc as plsc

M = 1_000_000
B = 16_384
NC = 2
NS = 16
HALF = M // NC
ACC = HALF + 8
CAT = 31_248
CH = 10_416
TAIL = 32
OB = B // NS
ZCH = 4_096


def _body(pm_hbm, pv_hbm, po_hbm, x_hbm, idx_hbm, out_hbm,
          idx_buf, x_buf, x2d, loc2d, ones2d, z_buf, po_buf,
          pm_buf, pv_buf, c_buf, s_buf, cnt_sh, sum_sh):
    c = lax.axis_index("c")
    s = lax.axis_index("s")

    pltpu.sync_copy(po_hbm, po_buf)
    pltpu.sync_copy(idx_hbm.at[pl.ds(s * OB, OB)], idx_buf)

    zeros16 = jnp.zeros((16,), jnp.float32)

    def zb(i, carry):
        z_buf[pl.ds(i * 16, 16)] = zeros16
        return carry

    lax.fori_loop(0, ZCH // 16, zb, 0)

    base = c * HALF
    ones16 = jnp.ones((16,), jnp.float32)

    def lb(i, carry):
        row = i >> 3
        col = (i & 7) * 16
        iv = idx_buf[pl.ds(i * 16, 16)]
        loc = iv - base
        in_core = (loc >= 0) & (loc < HALF)
        loc2d[row, pl.ds(col, 16)] = jnp.where(in_core, loc, HALF)
        ones2d[row, pl.ds(col, 16)] = ones16
        return carry

    lax.fori_loop(0, OB // 16, lb, 0)

    pltpu.sync_copy(x_hbm.at[pl.ds(s * OB, OB)], x_buf)

    def xb(i, carry):
        row = i >> 3
        col = (i & 7) * 16
        x2d[row, pl.ds(col, 16)] = x_buf[pl.ds(i * 16, 16)]
        return carry

    lax.fori_loop(0, OB // 16, xb, 0)

    zoff = s * CAT
    for k in range(CAT // ZCH):
        for sh in (cnt_sh, sum_sh):
            pltpu.sync_copy(z_buf, sh.at[pl.ds(zoff + k * ZCH, ZCH)])
    rem = CAT - (CAT // ZCH) * ZCH
    for sh in (cnt_sh, sum_sh):
        pltpu.sync_copy(z_buf.at[pl.ds(0, rem)], sh.at[pl.ds(zoff + (CAT // ZCH) * ZCH, rem)])

    @pl.when(s == NS - 1)
    def _ztail():
        zt = NS * CAT
        for sh in (cnt_sh, sum_sh):
            pltpu.sync_copy(z_buf.at[pl.ds(0, ACC - zt)], sh.at[pl.ds(zt, ACC - zt)])

    plsc.subcore_barrier()

    for j in range(OB // 128):
        pltpu.sync_copy(ones2d.at[j], cnt_sh.at[loc2d.at[j]], add=True)
        pltpu.sync_copy(x2d.at[j], sum_sh.at[loc2d.at[j]], add=True)

    plsc.subcore_barrier()

    po = po_buf[...]

    def compute(n_vecs):
        def body(i, carry):
            sl = pl.ds(i * 16, 16)
            pm = pm_buf[sl]
            pv = pv_buf[sl]
            cnt = c_buf[sl]
            sx = s_buf[sl]
            p0 = 1.0 / pv
            pn = p0 + cnt * po
            pm_buf[sl] = (pm * p0 + sx * po) / pn
            pv_buf[sl] = 1.0 / pn
            return carry
        lax.fori_loop(0, n_vecs, body, 0)

    cat0 = s * CAT
    for k in range(CAT // CH):
        off = cat0 + k * CH
        g = base + off
        pltpu.sync_copy(pm_hbm.at[pl.ds(g, CH)], pm_buf)
        pltpu.sync_copy(pv_hbm.at[pl.ds(g, CH)], pv_buf)
        pltpu.sync_copy(cnt_sh.at[pl.ds(off, CH)], c_buf)
        pltpu.sync_copy(sum_sh.at[pl.ds(off, CH)], s_buf)
        compute(CH // 16)
        pltpu.sync_copy(pm_buf, out_hbm.at[pl.ds(g, CH)])
        pltpu.sync_copy(pv_buf, out_hbm.at[pl.ds(M + g, CH)])

    @pl.when(s == NS - 1)
    def _tail():
        off = NS * CAT
        g = base + off
        pltpu.sync_copy(pm_hbm.at[pl.ds(g, TAIL)], pm_buf.at[pl.ds(0, TAIL)])
        pltpu.sync_copy(pv_hbm.at[pl.ds(g, TAIL)], pv_buf.at[pl.ds(0, TAIL)])
        pltpu.sync_copy(cnt_sh.at[pl.ds(off, TAIL)], c_buf.at[pl.ds(0, TAIL)])
        pltpu.sync_copy(sum_sh.at[pl.ds(off, TAIL)], s_buf.at[pl.ds(0, TAIL)])
        compute(TAIL // 16)
        pltpu.sync_copy(pm_buf.at[pl.ds(0, TAIL)], out_hbm.at[pl.ds(g, TAIL)])
        pltpu.sync_copy(pv_buf.at[pl.ds(0, TAIL)], out_hbm.at[pl.ds(M + g, TAIL)])


def kernel(prior_mean, prior_var, obs_variance, x, idx_tensor):
    po_vec = jnp.full((16,), 1.0, jnp.float32) / obs_variance

    mesh = plsc.VectorSubcoreMesh(core_axis_name="c", subcore_axis_name="s")
    run = pl.kernel(
        _body,
        out_type=jax.ShapeDtypeStruct((2 * M,), jnp.float32),
        mesh=mesh,
        scratch_types=(
            pltpu.VMEM((OB,), jnp.int32),
            pltpu.VMEM((OB,), jnp.float32),
            pltpu.VMEM((OB // 128, 128), jnp.float32),
            pltpu.VMEM((OB // 128, 128), jnp.int32),
            pltpu.VMEM((OB // 128, 128), jnp.float32),
            pltpu.VMEM((ZCH,), jnp.float32),
            pltpu.VMEM((16,), jnp.float32),
            pltpu.VMEM((CH,), jnp.float32),
            pltpu.VMEM((CH,), jnp.float32),
            pltpu.VMEM((CH,), jnp.float32),
            pltpu.VMEM((CH,), jnp.float32),
            pltpu.VMEM_SHARED((ACC,), jnp.float32),
            pltpu.VMEM_SHARED((ACC,), jnp.float32),
        ),
    )
    out = run(prior_mean, prior_var, po_vec, x, idx_tensor)
    return out.reshape(2, M)
